# Initial kernel scaffold; baseline (speedup 1.0000x reference)
import jax
import jax.numpy as jnp
from jax.experimental import pallas as pl


def kernel(cont_p, cont_c, cat_p, cat_c, val_len, diff_days, W1p, b1p, W2p, b2p, W1c, b1c, W2c, b2c, tab_gender, tab_korean, tab_primary, tab_job, tab_rep, tab_place, tab_add):
    raise NotImplementedError("write your pallas kernel here")


# speedup vs baseline: 8.1350x; 8.1350x over previous
import functools

import jax
import jax.numpy as jnp
from jax.experimental import pallas as pl
from jax.experimental.pallas import tpu as pltpu

_N_TOK = 4096 * 50
_EMB = 32
_R = 2048

_OFF_JOB, _OFF_REP, _OFF_PLACE, _OFF_ADD = 0, 11, 45, 64


def _tc_body(catp_ref, catc_ref, cp_ref, cc_ref, a1_ref, b1_ref,
             wcat_ref, a2_ref, b2_ref, out_ref):
    catp = catp_ref[...]
    catc = catc_ref[...]

    x1 = jnp.concatenate(
        [catp[:, 0:3].astype(jnp.float32), cp_ref[...], cc_ref[...]], axis=1)
    y1 = jnp.dot(x1, a1_ref[...], preferred_element_type=jnp.float32) \
        + b1_ref[...]
    left_base = y1[:, 0:_EMB]
    hp = y1[:, _EMB:2 * _EMB]
    hc = y1[:, 2 * _EMB:3 * _EMB]
    hp = jnp.where(hp > 0, hp, jnp.exp(hp) - 1.0)
    hc = jnp.where(hc > 0, hc, jnp.exp(hc) - 1.0)

    col = jax.lax.broadcasted_iota(jnp.int32, (catp.shape[0], 128), 1)
    m = (col == catp[:, 3:4] + _OFF_JOB)
    m = m | (col == catp[:, 4:5] + _OFF_REP)
    m = m | (col == catc[:, 0:1] + _OFF_PLACE)
    m = m | (col == catc[:, 1:2] + _OFF_ADD)
    oh = m.astype(jnp.float32)
    y0 = jnp.dot(oh, wcat_ref[...], preferred_element_type=jnp.float32)

    y2 = jnp.dot(jnp.concatenate([hp, hc], axis=1), a2_ref[...],
                 preferred_element_type=jnp.float32) + b2_ref[...]

    out_ref[:, 0:_EMB] = y0[:, 0:_EMB] + left_base
    out_ref[:, _EMB:2 * _EMB] = y0[:, _EMB:2 * _EMB]
    out_ref[:, 2 * _EMB:] = y2


@functools.partial(jax.jit, static_argnames=())
def _run(catp, catc, cp, cc, a1, b1, wcat, a2, b2):
    grid = (_N_TOK // _R,)
    row = lambda i: (i, 0)
    fixed = lambda i: (0, 0)
    return pl.pallas_call(
        _tc_body,
        grid=grid,
        in_specs=[
            pl.BlockSpec((_R, 5), row),
            pl.BlockSpec((_R, 2), row),
            pl.BlockSpec((_R, 3), row),
            pl.BlockSpec((_R, 2), row),
            pl.BlockSpec((8, 96), fixed),
            pl.BlockSpec((1, 96), fixed),
            pl.BlockSpec((128, 64), fixed),
            pl.BlockSpec((64, 64), fixed),
            pl.BlockSpec((1, 64), fixed),
        ],
        out_specs=pl.BlockSpec((_R, 128), row),
        out_shape=jax.ShapeDtypeStruct((_N_TOK, 128), jnp.float32),
        compiler_params=pltpu.CompilerParams(
            dimension_semantics=("arbitrary",)),
    )(catp, catc, cp, cc, a1, b1, wcat, a2, b2)


def kernel(cont_p, cont_c, cat_p, cat_c, val_len, diff_days,
           W1p, b1p, W2p, b2p, W1c, b1c, W2c, b2c,
           tab_gender, tab_korean, tab_primary, tab_job, tab_rep,
           tab_place, tab_add):
    B, L = cont_p.shape[0], cont_p.shape[1]

    catp = cat_p.astype(jnp.int32).reshape(_N_TOK, 5)
    catc = cat_c.astype(jnp.int32).reshape(_N_TOK, 2)
    cp = cont_p.reshape(_N_TOK, 3)
    cc = cont_c.reshape(_N_TOK, 2)

    g0 = (tab_gender[0] + tab_korean[0] + tab_primary[0]) / 5.0
    gd = (tab_gender[1] - tab_gender[0]) / 5.0
    kd = (tab_korean[1] - tab_korean[0]) / 5.0
    pd = (tab_primary[1] - tab_primary[0]) / 5.0

    a1 = jnp.zeros((8, 96), jnp.float32)
    a1 = a1.at[0, 0:_EMB].set(gd).at[1, 0:_EMB].set(kd).at[2, 0:_EMB].set(pd)
    a1 = a1.at[3:6, _EMB:2 * _EMB].set(W1p)
    a1 = a1.at[6:8, 2 * _EMB:3 * _EMB].set(W1c)
    b1 = jnp.concatenate([g0, b1p, b1c]).reshape(1, 96)

    wcat = jnp.zeros((128, 64), jnp.float32)
    wcat = wcat.at[_OFF_JOB:_OFF_JOB + 11, 0:_EMB].set(tab_job / 5.0)
    wcat = wcat.at[_OFF_REP:_OFF_REP + 34, 0:_EMB].set(tab_rep / 5.0)
    wcat = wcat.at[_OFF_PLACE:_OFF_PLACE + 19, _EMB:].set(tab_place / 2.0)
    wcat = wcat.at[_OFF_ADD:_OFF_ADD + 31, _EMB:].set(tab_add / 2.0)

    a2 = jnp.zeros((64, 64), jnp.float32)
    a2 = a2.at[0:_EMB, 0:_EMB].set(W2p).at[_EMB:, _EMB:].set(W2c)
    b2 = jnp.concatenate([b2p, b2c]).reshape(1, 64)

    x = _run(catp, catc, cp, cc, a1, b1, wcat, a2, b2)
    return (x.reshape(B, L, 128), diff_days, val_len)


# speedup vs baseline: 33.3435x; 4.0988x over previous
import jax
import jax.numpy as jnp
from jax.experimental import pallas as pl
from jax.experimental.pallas import tpu as pltpu

_B, _L = 4096, 50
_EMB = 32

_OFF_JOB, _OFF_REP, _OFF_PLACE, _OFF_ADD = 0, 11, 45, 64

_TDOT = (((0,), (0,)), ((), ()))


def _tc_body(p0, p1, p2, q0, q1, q2, r0, r1, p3, p4, c0, c1,
             a1_ref, b1_ref, wcat_ref, a2_ref, b2_ref, out_ref):
    x1t = jnp.concatenate(
        [p0[0], p1[0], p2[0], q0[0], q1[0], q2[0], r0[0], r1[0]],
        axis=0)
    y1 = jax.lax.dot_general(x1t, a1_ref[...], _TDOT,
                             preferred_element_type=jnp.float32) \
        + b1_ref[...]
    left_base = y1[:, 0:_EMB]
    hp = y1[:, _EMB:2 * _EMB]
    hc = y1[:, 2 * _EMB:3 * _EMB]
    hp = jnp.where(hp > 0, hp, jnp.exp(hp) - 1.0)
    hc = jnp.where(hc > 0, hc, jnp.exp(hc) - 1.0)

    row = jax.lax.broadcasted_iota(jnp.int32, (128, _B), 0)
    m = (row == p3[0] + _OFF_JOB)
    m = m | (row == p4[0] + _OFF_REP)
    m = m | (row == c0[0] + _OFF_PLACE)
    m = m | (row == c1[0] + _OFF_ADD)
    oht = m.astype(jnp.float32)
    y0 = jax.lax.dot_general(oht, wcat_ref[...], _TDOT,
                             preferred_element_type=jnp.float32)

    y2 = jnp.dot(jnp.concatenate([hp, hc], axis=1), a2_ref[...],
                 preferred_element_type=jnp.float32) + b2_ref[...]

    out_ref[0, :, 0:_EMB] = y0[:, 0:_EMB] + left_base
    out_ref[0, :, _EMB:2 * _EMB] = y0[:, _EMB:2 * _EMB]
    out_ref[0, :, 2 * _EMB:] = y2


@jax.jit
def _run(planes, a1, b1, wcat, a2, b2):
    plane_spec = pl.BlockSpec((1, 1, _B), lambda l: (l, 0, 0))
    fixed = lambda l: (0, 0)
    return pl.pallas_call(
        _tc_body,
        grid=(_L,),
        in_specs=[plane_spec] * 12 + [
            pl.BlockSpec((8, 96), fixed),
            pl.BlockSpec((1, 96), fixed),
            pl.BlockSpec((128, 64), fixed),
            pl.BlockSpec((64, 64), fixed),
            pl.BlockSpec((1, 64), fixed),
        ],
        out_specs=pl.BlockSpec((1, _B, 128), lambda l: (l, 0, 0)),
        out_shape=jax.ShapeDtypeStruct((_L, _B, 128), jnp.float32),
        compiler_params=pltpu.CompilerParams(
            dimension_semantics=("arbitrary",)),
    )(*planes, a1, b1, wcat, a2, b2)


def kernel(cont_p, cont_c, cat_p, cat_c, val_len, diff_days,
           W1p, b1p, W2p, b2p, W1c, b1c, W2c, b2c,
           tab_gender, tab_korean, tab_primary, tab_job, tab_rep,
           tab_place, tab_add):
    def fplane(arr, j):
        return jnp.transpose(arr[:, :, j]).reshape(_L, 1, _B)

    catp = cat_p.astype(jnp.int32)
    catc = cat_c.astype(jnp.int32)
    planes = (
        fplane(catp, 0).astype(jnp.float32),
        fplane(catp, 1).astype(jnp.float32),
        fplane(catp, 2).astype(jnp.float32),
        fplane(cont_p, 0), fplane(cont_p, 1), fplane(cont_p, 2),
        fplane(cont_c, 0), fplane(cont_c, 1),
        fplane(catp, 3), fplane(catp, 4),
        fplane(catc, 0), fplane(catc, 1),
    )

    g0 = (tab_gender[0] + tab_korean[0] + tab_primary[0]) / 5.0
    gd = (tab_gender[1] - tab_gender[0]) / 5.0
    kd = (tab_korean[1] - tab_korean[0]) / 5.0
    pd = (tab_primary[1] - tab_primary[0]) / 5.0

    a1 = jnp.zeros((8, 96), jnp.float32)
    a1 = a1.at[0, 0:_EMB].set(gd).at[1, 0:_EMB].set(kd).at[2, 0:_EMB].set(pd)
    a1 = a1.at[3:6, _EMB:2 * _EMB].set(W1p)
    a1 = a1.at[6:8, 2 * _EMB:3 * _EMB].set(W1c)
    b1 = jnp.concatenate([g0, b1p, b1c]).reshape(1, 96)

    wcat = jnp.zeros((128, 64), jnp.float32)
    wcat = wcat.at[_OFF_JOB:_OFF_JOB + 11, 0:_EMB].set(tab_job / 5.0)
    wcat = wcat.at[_OFF_REP:_OFF_REP + 34, 0:_EMB].set(tab_rep / 5.0)
    wcat = wcat.at[_OFF_PLACE:_OFF_PLACE + 19, _EMB:].set(tab_place / 2.0)
    wcat = wcat.at[_OFF_ADD:_OFF_ADD + 31, _EMB:].set(tab_add / 2.0)

    a2 = jnp.zeros((64, 64), jnp.float32)
    a2 = a2.at[0:_EMB, 0:_EMB].set(W2p).at[_EMB:, _EMB:].set(W2c)
    b2 = jnp.concatenate([b2p, b2c]).reshape(1, 64)

    y = _run(planes, a1, b1, wcat, a2, b2)
    x = jnp.transpose(y, (1, 0, 2))
    return (x, diff_days, val_len)


# speedup vs baseline: 46.0493x; 1.3811x over previous
import jax
import jax.numpy as jnp
from jax.experimental import pallas as pl
from jax.experimental.pallas import tpu as pltpu

_B, _L = 4096, 50
_EMB = 32

_OFF_JOB, _OFF_REP, _OFF_PLACE, _OFF_ADD = 0, 11, 45, 64

_TDOT = (((0,), (0,)), ((), ()))


def _tc_body(p0, p1, p2, q0, q1, q2, r0, r1, p3, p4, c0, c1,
             a1_ref, b1_ref, wall_ref, ball_ref, out_ref):
    x1t = jnp.concatenate(
        [p0[0], p1[0], p2[0], q0[0], q1[0], q2[0], r0[0], r1[0]],
        axis=0)
    y1t = jax.lax.dot_general(a1_ref[...], x1t, _TDOT,
                              preferred_element_type=jnp.float32) \
        + b1_ref[...]
    leftt = y1t[0:_EMB, :]
    hp = y1t[_EMB:2 * _EMB, :]
    hc = y1t[2 * _EMB:3 * _EMB, :]
    hp = jnp.where(hp > 0, hp, jnp.exp(hp) - 1.0)
    hc = jnp.where(hc > 0, hc, jnp.exp(hc) - 1.0)

    row = jax.lax.broadcasted_iota(jnp.int32, (96, _B), 0)
    m = (row == p3[0] + _OFF_JOB)
    m = m | (row == p4[0] + _OFF_REP)
    m = m | (row == c0[0] + _OFF_PLACE)
    m = m | (row == c1[0] + _OFF_ADD)
    oht = m.astype(jnp.float32)

    lhs = jnp.concatenate([oht, hp, hc, leftt], axis=0)
    y = jax.lax.dot_general(lhs, wall_ref[...], _TDOT,
                            preferred_element_type=jnp.float32) \
        + ball_ref[...]
    out_ref[0] = y


@jax.jit
def _run(planes, a1, b1, wall, ball):
    plane_spec = pl.BlockSpec((1, 1, _B), lambda l: (l, 0, 0))
    fixed = lambda l: (0, 0)
    return pl.pallas_call(
        _tc_body,
        grid=(_L,),
        in_specs=[plane_spec] * 12 + [
            pl.BlockSpec((8, 96), fixed),
            pl.BlockSpec((96, 1), fixed),
            pl.BlockSpec((192, 128), fixed),
            pl.BlockSpec((1, 128), fixed),
        ],
        out_specs=pl.BlockSpec((1, _B, 128), lambda l: (l, 0, 0)),
        out_shape=jax.ShapeDtypeStruct((_L, _B, 128), jnp.float32),
        compiler_params=pltpu.CompilerParams(
            dimension_semantics=("arbitrary",)),
    )(*planes, a1, b1, wall, ball)


def kernel(cont_p, cont_c, cat_p, cat_c, val_len, diff_days,
           W1p, b1p, W2p, b2p, W1c, b1c, W2c, b2c,
           tab_gender, tab_korean, tab_primary, tab_job, tab_rep,
           tab_place, tab_add):
    def fplane(arr, j):
        return jnp.transpose(arr[:, :, j]).reshape(_L, 1, _B)

    catp = cat_p.astype(jnp.int32)
    catc = cat_c.astype(jnp.int32)
    planes = (
        fplane(catp, 0).astype(jnp.float32),
        fplane(catp, 1).astype(jnp.float32),
        fplane(catp, 2).astype(jnp.float32),
        fplane(cont_p, 0), fplane(cont_p, 1), fplane(cont_p, 2),
        fplane(cont_c, 0), fplane(cont_c, 1),
        fplane(catp, 3), fplane(catp, 4),
        fplane(catc, 0), fplane(catc, 1),
    )

    g0 = (tab_gender[0] + tab_korean[0] + tab_primary[0]) / 5.0
    gd = (tab_gender[1] - tab_gender[0]) / 5.0
    kd = (tab_korean[1] - tab_korean[0]) / 5.0
    pd = (tab_primary[1] - tab_primary[0]) / 5.0

    a1 = jnp.zeros((8, 96), jnp.float32)
    a1 = a1.at[0, 0:_EMB].set(gd).at[1, 0:_EMB].set(kd).at[2, 0:_EMB].set(pd)
    a1 = a1.at[3:6, _EMB:2 * _EMB].set(W1p)
    a1 = a1.at[6:8, 2 * _EMB:3 * _EMB].set(W1c)
    b1 = jnp.concatenate([g0, b1p, b1c]).reshape(96, 1)

    wall = jnp.zeros((192, 128), jnp.float32)
    wall = wall.at[_OFF_JOB:_OFF_JOB + 11, 0:_EMB].set(tab_job / 5.0)
    wall = wall.at[_OFF_REP:_OFF_REP + 34, 0:_EMB].set(tab_rep / 5.0)
    wall = wall.at[_OFF_PLACE:_OFF_PLACE + 19, _EMB:2 * _EMB].set(
        tab_place / 2.0)
    wall = wall.at[_OFF_ADD:_OFF_ADD + 31, _EMB:2 * _EMB].set(tab_add / 2.0)
    wall = wall.at[96:128, 2 * _EMB:3 * _EMB].set(W2p)
    wall = wall.at[128:160, 3 * _EMB:].set(W2c)
    wall = wall.at[160:192, 0:_EMB].set(jnp.eye(_EMB, dtype=jnp.float32))
    ball = jnp.concatenate(
        [jnp.zeros((2 * _EMB,), jnp.float32), b2p, b2c]).reshape(1, 128)

    y = _run(planes, a1, b1, wall, ball)
    x = jnp.transpose(y, (1, 0, 2))
    return (x, diff_days, val_len)


# speedup vs baseline: 46.1737x; 1.0027x over previous
import jax
import jax.numpy as jnp
from jax.experimental import pallas as pl
from jax.experimental.pallas import tpu as pltpu

_B, _L = 4096, 50
_EMB = 32

_OFF_JOB, _OFF_REP, _OFF_PLACE, _OFF_ADD = 0, 11, 45, 64

_TDOT = (((0,), (0,)), ((), ()))


def _tc_body(p0, p1, p2, q0, q1, q2, r0, r1, p3, p4, c0, c1,
             a1_ref, wall_ref, ball_ref, out_ref):
    ones = jnp.ones((1, _B), jnp.float32)
    x1t = jnp.concatenate(
        [p0[0], p1[0], p2[0], q0[0], q1[0], q2[0], r0[0], r1[0], ones],
        axis=0)
    y1t = jax.lax.dot_general(a1_ref[...], x1t, _TDOT,
                              preferred_element_type=jnp.float32)

    row = jax.lax.broadcasted_iota(jnp.int32, (96, _B), 0)
    y2t = jnp.where((y1t > 0) | (row >= 64), y1t, jnp.exp(y1t) - 1.0)

    m = (row == p3[0] + _OFF_JOB)
    m = m | (row == p4[0] + _OFF_REP)
    m = m | (row == c0[0] + _OFF_PLACE)
    m = m | (row == c1[0] + _OFF_ADD)
    oht = m.astype(jnp.float32)

    lhs = jnp.concatenate([oht, y2t], axis=0)
    y = jax.lax.dot_general(lhs, wall_ref[...], _TDOT,
                            preferred_element_type=jnp.float32) \
        + ball_ref[...]
    out_ref[0] = y


@jax.jit
def _run(planes, a1, wall, ball):
    plane_spec = pl.BlockSpec((1, 1, _B), lambda l: (l, 0, 0))
    fixed = lambda l: (0, 0)
    return pl.pallas_call(
        _tc_body,
        grid=(_L,),
        in_specs=[plane_spec] * 12 + [
            pl.BlockSpec((9, 96), fixed),
            pl.BlockSpec((192, 128), fixed),
            pl.BlockSpec((1, 128), fixed),
        ],
        out_specs=pl.BlockSpec((1, _B, 128), lambda l: (l, 0, 0)),
        out_shape=jax.ShapeDtypeStruct((_L, _B, 128), jnp.float32),
        compiler_params=pltpu.CompilerParams(
            dimension_semantics=("arbitrary",)),
    )(*planes, a1, wall, ball)


def kernel(cont_p, cont_c, cat_p, cat_c, val_len, diff_days,
           W1p, b1p, W2p, b2p, W1c, b1c, W2c, b2c,
           tab_gender, tab_korean, tab_primary, tab_job, tab_rep,
           tab_place, tab_add):
    def fplane(arr, j):
        return jnp.transpose(arr[:, :, j]).reshape(_L, 1, _B)

    catp = cat_p.astype(jnp.int32)
    catc = cat_c.astype(jnp.int32)
    planes = (
        fplane(catp, 0).astype(jnp.float32),
        fplane(catp, 1).astype(jnp.float32),
        fplane(catp, 2).astype(jnp.float32),
        fplane(cont_p, 0), fplane(cont_p, 1), fplane(cont_p, 2),
        fplane(cont_c, 0), fplane(cont_c, 1),
        fplane(catp, 3), fplane(catp, 4),
        fplane(catc, 0), fplane(catc, 1),
    )

    g0 = (tab_gender[0] + tab_korean[0] + tab_primary[0]) / 5.0
    gd = (tab_gender[1] - tab_gender[0]) / 5.0
    kd = (tab_korean[1] - tab_korean[0]) / 5.0
    pd = (tab_primary[1] - tab_primary[0]) / 5.0

    a1 = jnp.zeros((9, 96), jnp.float32)
    a1 = a1.at[3:6, 0:_EMB].set(W1p)
    a1 = a1.at[6:8, _EMB:2 * _EMB].set(W1c)
    a1 = a1.at[0, 2 * _EMB:].set(gd).at[1, 2 * _EMB:].set(kd)
    a1 = a1.at[2, 2 * _EMB:].set(pd)
    a1 = a1.at[8, 0:_EMB].set(b1p).at[8, _EMB:2 * _EMB].set(b1c)
    a1 = a1.at[8, 2 * _EMB:].set(g0)

    wall = jnp.zeros((192, 128), jnp.float32)
    wall = wall.at[_OFF_JOB:_OFF_JOB + 11, 0:_EMB].set(tab_job / 5.0)
    wall = wall.at[_OFF_REP:_OFF_REP + 34, 0:_EMB].set(tab_rep / 5.0)
    wall = wall.at[_OFF_PLACE:_OFF_PLACE + 19, _EMB:2 * _EMB].set(
        tab_place / 2.0)
    wall = wall.at[_OFF_ADD:_OFF_ADD + 31, _EMB:2 * _EMB].set(tab_add / 2.0)
    wall = wall.at[96:128, 2 * _EMB:3 * _EMB].set(W2p)
    wall = wall.at[128:160, 3 * _EMB:].set(W2c)
    wall = wall.at[160:192, 0:_EMB].set(jnp.eye(_EMB, dtype=jnp.float32))
    ball = jnp.concatenate(
        [jnp.zeros((2 * _EMB,), jnp.float32), b2p, b2c]).reshape(1, 128)

    y = _run(planes, a1, wall, ball)
    x = jnp.transpose(y, (1, 0, 2))
    return (x, diff_days, val_len)
